# baseline (speedup 1.0000x reference)
import functools

import jax
import jax.numpy as jnp
from jax import lax
from jax.experimental import pallas as pl
from jax.experimental.pallas import tpu as pltpu
from jax.experimental.pallas import tpu_sc as plsc

_P = 4


def _sc_gather4(uidx4, iidx4, t_umf, t_imf, t_umlp, t_imlp):
    B = uidx4.shape[0]
    D = t_umf.shape[1]
    info = plsc.get_sparse_core_info()
    nc = info.num_cores
    nw = nc * info.num_subcores
    bpw = B // nw
    mesh = plsc.VectorSubcoreMesh(core_axis_name="c", subcore_axis_name="s")
    out_t = jax.ShapeDtypeStruct((B, D), jnp.float32)

    @functools.partial(
        pl.kernel,
        mesh=mesh,
        out_type=[out_t, out_t, out_t, out_t],
        scratch_types=[
            pltpu.VMEM((bpw,), jnp.int32),
            pltpu.VMEM((bpw,), jnp.int32),
            pltpu.VMEM((bpw, D), jnp.float32),
            pltpu.SemaphoreType.DMA,
        ],
    )
    def gather_kernel(uidx_hbm, iidx_hbm, umf, imf, umlp, imlp,
                      o_umf, o_imf, o_umlp, o_imlp,
                      sidx_u, sidx_i, rows, gsem):
        wid = lax.axis_index("s") * nc + lax.axis_index("c")
        base = wid * bpw
        sl = pl.ds(base, bpw)
        pltpu.sync_copy(uidx_hbm.at[sl], sidx_u)
        pltpu.sync_copy(iidx_hbm.at[sl], sidx_i)

        def do_table(ti, tbl, sidx, oref):
            def fire(j, _):
                v = sidx[pl.ds(j * 16, 16)]
                for l in range(16):
                    pltpu.async_copy(tbl.at[v[l]], rows.at[j * 16 + l], gsem)
                return _

            lax.fori_loop(0, bpw // 16, fire, None)

            def drain(j, _):
                pltpu.make_async_copy(tbl.at[0], rows.at[0], gsem).wait()
                return _

            lax.fori_loop(0, bpw, drain, None)
            pltpu.sync_copy(rows, oref.at[sl])

        do_table(0, umf, sidx_u, o_umf)
        do_table(1, imf, sidx_i, o_imf)
        do_table(2, umlp, sidx_u, o_umlp)
        do_table(3, imlp, sidx_i, o_imlp)

    return gather_kernel(uidx4, iidx4, t_umf, t_imf, t_umlp, t_imlp)


def _tc_mlp(u_mf4, i_mf4, u_mlp4, i_mlp4, oh_u, oh_i,
            W1, b1, W2, b2, W3, b3, Wo, bo):
    B = u_mf4.shape[0]
    D = 32
    BLK = 2048
    grid = B // BLK
    w1a, w1b = W1[:D], W1[D:]
    womf_t = Wo[:D].reshape(1, D)
    woh_t = Wo[D:].reshape(1, -1)
    b1r = b1.reshape(1, -1)
    b2r = b2.reshape(1, -1)
    b3r = b3.reshape(1, -1)
    bor = bo.reshape(1, 1)

    def sel(x4, oh):
        acc = oh[:, 0:1] * x4[:, 0:D]
        for k in range(1, _P):
            acc = acc + oh[:, k:k + 1] * x4[:, k * D:(k + 1) * D]
        return acc

    def body(umf_ref, imf_ref, umlp_ref, imlp_ref, ohu_ref, ohi_ref,
             w1a_ref, w1b_ref, b1_ref, w2_ref, b2_ref, w3_ref, b3_ref,
             womf_ref, woh_ref, bo_ref, out_ref):
        ohu = ohu_ref[...]
        ohi = ohi_ref[...]
        u_mlp = sel(umlp_ref[...], ohu)
        i_mlp = sel(imlp_ref[...], ohi)
        h = jnp.dot(u_mlp, w1a_ref[...], preferred_element_type=jnp.float32)
        h = h + jnp.dot(i_mlp, w1b_ref[...], preferred_element_type=jnp.float32)
        h = jnp.maximum(h + b1_ref[...], 0.0)
        h = jnp.dot(h, w2_ref[...], preferred_element_type=jnp.float32)
        h = jnp.maximum(h + b2_ref[...], 0.0)
        h = jnp.dot(h, w3_ref[...], preferred_element_type=jnp.float32)
        h = jnp.maximum(h + b3_ref[...], 0.0)
        mf = sel(umf_ref[...], ohu) * sel(imf_ref[...], ohi)
        acc = mf * womf_ref[...] + h * woh_ref[...]
        out_ref[...] = jnp.sum(acc, axis=1, keepdims=True) + bo_ref[...]

    row_spec = pl.BlockSpec((BLK, _P * D), lambda i: (i, 0))
    oh_spec = pl.BlockSpec((BLK, _P), lambda i: (i, 0))
    full = lambda a: pl.BlockSpec(a.shape, lambda i: (0,) * a.ndim)
    out = pl.pallas_call(
        body,
        grid=(grid,),
        in_specs=[row_spec, row_spec, row_spec, row_spec, oh_spec, oh_spec,
                  full(w1a), full(w1b), full(b1r), full(W2), full(b2r),
                  full(W3), full(b3r), full(womf_t), full(woh_t), full(bor)],
        out_specs=pl.BlockSpec((BLK, 1), lambda i: (i, 0)),
        out_shape=jax.ShapeDtypeStruct((B, 1), jnp.float32),
    )(u_mf4, i_mf4, u_mlp4, i_mlp4, oh_u, oh_i,
      w1a, w1b, b1r, W2, b2r, W3, b3r, womf_t, woh_t, bor)
    return out[:, 0]


def kernel(user_idx, item_idx, user_embedding_mf, item_embedding_mf,
           user_embedding_mlp, item_embedding_mlp, W1, b1, W2, b2, W3, b3,
           Wo, bo):
    V, D = user_embedding_mf.shape
    uidx = user_idx.astype(jnp.int32)
    iidx = item_idx.astype(jnp.int32)
    pack = lambda t: t.reshape(V // _P, _P * D)
    u_mf4, i_mf4, u_mlp4, i_mlp4 = _sc_gather4(
        uidx // _P, iidx // _P,
        pack(user_embedding_mf), pack(item_embedding_mf),
        pack(user_embedding_mlp), pack(item_embedding_mlp))
    oh_u = jax.nn.one_hot(uidx % _P, _P, dtype=jnp.float32)
    oh_i = jax.nn.one_hot(iidx % _P, _P, dtype=jnp.float32)
    return _tc_mlp(u_mf4, i_mf4, u_mlp4, i_mlp4, oh_u, oh_i,
                   W1, b1, W2, b2, W3, b3, Wo, bo)


# speedup vs baseline: 1.2137x; 1.2137x over previous
import functools

import jax
import jax.numpy as jnp
from jax import lax
from jax.experimental import pallas as pl
from jax.experimental.pallas import tpu as pltpu
from jax.experimental.pallas import tpu_sc as plsc


def _sc_gather2(uidx, iidx, t_all):
    B = uidx.shape[0]
    DF = t_all.shape[1]
    info = plsc.get_sparse_core_info()
    nc = info.num_cores
    nw = nc * info.num_subcores
    bpw = B // nw
    mesh = plsc.VectorSubcoreMesh(core_axis_name="c", subcore_axis_name="s")
    out_t = jax.ShapeDtypeStruct((B, DF), jnp.float32)

    @functools.partial(
        pl.kernel,
        mesh=mesh,
        out_type=[out_t, out_t],
        scratch_types=[
            pltpu.VMEM((bpw,), jnp.int32),
            pltpu.VMEM((bpw,), jnp.int32),
            pltpu.VMEM((bpw, DF), jnp.float32),
            pltpu.SemaphoreType.DMA,
        ],
    )
    def gather_kernel(uidx_hbm, iidx_hbm, tbl, o_u, o_i,
                      sidx_u, sidx_i, rows, gsem):
        wid = lax.axis_index("s") * nc + lax.axis_index("c")
        base = wid * bpw
        sl = pl.ds(base, bpw)
        pltpu.sync_copy(uidx_hbm.at[sl], sidx_u)
        pltpu.sync_copy(iidx_hbm.at[sl], sidx_i)

        def do_side(sidx, oref):
            def fire(j, _):
                v = sidx[pl.ds(j * 16, 16)]
                for l in range(16):
                    pltpu.async_copy(tbl.at[v[l]], rows.at[j * 16 + l], gsem)
                return _

            lax.fori_loop(0, bpw // 16, fire, None)

            def drain(j, _):
                pltpu.make_async_copy(tbl.at[0], rows.at[0], gsem).wait()
                return _

            lax.fori_loop(0, bpw, drain, None)
            pltpu.sync_copy(rows, oref.at[sl])

        do_side(sidx_u, o_u)
        do_side(sidx_i, o_i)

    return gather_kernel(uidx, iidx, t_all)


def _tc_mlp(u_rows, i_rows, W1, b1, W2, b2, W3, b3, Wo, bo):
    B, DF = u_rows.shape
    D = DF // 4
    BLK = 2048
    grid = B // BLK
    w1a, w1b = W1[:D], W1[D:]
    womf_t = Wo[:D].reshape(1, D)
    woh_t = Wo[D:].reshape(1, -1)
    b1r = b1.reshape(1, -1)
    b2r = b2.reshape(1, -1)
    b3r = b3.reshape(1, -1)
    bor = bo.reshape(1, 1)

    def body(u_ref, i_ref,
             w1a_ref, w1b_ref, b1_ref, w2_ref, b2_ref, w3_ref, b3_ref,
             womf_ref, woh_ref, bo_ref, out_ref):
        u = u_ref[...]
        i = i_ref[...]
        u_mf = u[:, 0:D]
        i_mf = i[:, D:2 * D]
        u_mlp = u[:, 2 * D:3 * D]
        i_mlp = i[:, 3 * D:4 * D]
        h = jnp.dot(u_mlp, w1a_ref[...], preferred_element_type=jnp.float32)
        h = h + jnp.dot(i_mlp, w1b_ref[...], preferred_element_type=jnp.float32)
        h = jnp.maximum(h + b1_ref[...], 0.0)
        h = jnp.dot(h, w2_ref[...], preferred_element_type=jnp.float32)
        h = jnp.maximum(h + b2_ref[...], 0.0)
        h = jnp.dot(h, w3_ref[...], preferred_element_type=jnp.float32)
        h = jnp.maximum(h + b3_ref[...], 0.0)
        mf = u_mf * i_mf
        acc = mf * womf_ref[...] + h * woh_ref[...]
        out_ref[...] = jnp.sum(acc, axis=1, keepdims=True) + bo_ref[...]

    row_spec = pl.BlockSpec((BLK, DF), lambda i: (i, 0))
    full = lambda a: pl.BlockSpec(a.shape, lambda i: (0,) * a.ndim)
    out = pl.pallas_call(
        body,
        grid=(grid,),
        in_specs=[row_spec, row_spec,
                  full(w1a), full(w1b), full(b1r), full(W2), full(b2r),
                  full(W3), full(b3r), full(womf_t), full(woh_t), full(bor)],
        out_specs=pl.BlockSpec((BLK, 1), lambda i: (i, 0)),
        out_shape=jax.ShapeDtypeStruct((B, 1), jnp.float32),
    )(u_rows, i_rows, w1a, w1b, b1r, W2, b2r, W3, b3r, womf_t, woh_t, bor)
    return out[:, 0]


def kernel(user_idx, item_idx, user_embedding_mf, item_embedding_mf,
           user_embedding_mlp, item_embedding_mlp, W1, b1, W2, b2, W3, b3,
           Wo, bo):
    t_all = jnp.concatenate(
        [user_embedding_mf, item_embedding_mf,
         user_embedding_mlp, item_embedding_mlp], axis=1)
    u_rows, i_rows = _sc_gather2(
        user_idx.astype(jnp.int32), item_idx.astype(jnp.int32), t_all)
    return _tc_mlp(u_rows, i_rows, W1, b1, W2, b2, W3, b3, Wo, bo)


# speedup vs baseline: 1.2141x; 1.0003x over previous
import functools

import jax
import jax.numpy as jnp
from jax import lax
from jax.experimental import pallas as pl
from jax.experimental.pallas import tpu as pltpu
from jax.experimental.pallas import tpu_sc as plsc


def _sc_gather2(uidx, iidx, t_all):
    B = uidx.shape[0]
    DF = t_all.shape[1]
    info = plsc.get_sparse_core_info()
    nc = info.num_cores
    nw = nc * info.num_subcores
    bpw = B // nw
    mesh = plsc.VectorSubcoreMesh(core_axis_name="c", subcore_axis_name="s")
    out_t = jax.ShapeDtypeStruct((B, DF), jnp.float32)

    @functools.partial(
        pl.kernel,
        mesh=mesh,
        out_type=[out_t, out_t],
        scratch_types=[
            pltpu.VMEM((bpw,), jnp.int32),
            pltpu.VMEM((bpw,), jnp.int32),
            pltpu.VMEM((bpw, DF), jnp.float32),
            pltpu.SemaphoreType.DMA,
        ],
    )
    def gather_kernel(uidx_hbm, iidx_hbm, tbl, o_u, o_i,
                      sidx_u, sidx_i, rows, gsem):
        wid = lax.axis_index("s") * nc + lax.axis_index("c")
        base = wid * bpw
        sl = pl.ds(base, bpw)
        pltpu.sync_copy(uidx_hbm.at[sl], sidx_u)
        pltpu.sync_copy(iidx_hbm.at[sl], sidx_i)

        def do_side(sidx, oref):
            def fire(j, _):
                v = sidx[pl.ds(j * 16, 16)]
                for l in range(16):
                    pltpu.async_copy(tbl.at[v[l]], rows.at[j * 16 + l], gsem)
                return _

            lax.fori_loop(0, bpw // 16, fire, None)

            def drain(j, _):
                pltpu.make_async_copy(tbl.at[0], rows.at[0], gsem).wait()
                return _

            lax.fori_loop(0, bpw, drain, None)
            pltpu.sync_copy(rows, oref.at[sl])

        do_side(sidx_u, o_u)
        do_side(sidx_i, o_i)

    return gather_kernel(uidx, iidx, t_all)


def _tc_mlp(u_rows, i_rows, W1, b1, W2, b2, W3, b3, Wo, bo):
    B, DF = u_rows.shape
    D = DF // 4
    BLK = 2048
    grid = B // BLK
    w1a, w1b = W1[:D], W1[D:]
    womf_t = Wo[:D].reshape(1, D)
    woh_t = Wo[D:].reshape(1, -1)
    b1r = b1.reshape(1, -1)
    b2r = b2.reshape(1, -1)
    b3r = b3.reshape(1, -1)
    bor = bo.reshape(1, 1)

    def body(u_ref, i_ref,
             w1a_ref, w1b_ref, b1_ref, w2_ref, b2_ref, w3_ref, b3_ref,
             womf_ref, woh_ref, bo_ref, out_ref):
        u = u_ref[...]
        i = i_ref[...]
        u_mf = u[:, 0:D]
        i_mf = i[:, D:2 * D]
        u_mlp = u[:, 2 * D:3 * D]
        i_mlp = i[:, 3 * D:4 * D]
        h = jnp.dot(u_mlp, w1a_ref[...], preferred_element_type=jnp.float32)
        h = h + jnp.dot(i_mlp, w1b_ref[...], preferred_element_type=jnp.float32)
        h = jnp.maximum(h + b1_ref[...], 0.0)
        h = jnp.dot(h, w2_ref[...], preferred_element_type=jnp.float32)
        h = jnp.maximum(h + b2_ref[...], 0.0)
        h = jnp.dot(h, w3_ref[...], preferred_element_type=jnp.float32)
        h = jnp.maximum(h + b3_ref[...], 0.0)
        mf = u_mf * i_mf
        acc = mf * womf_ref[...] + h * woh_ref[...]
        out_ref[...] = jnp.sum(acc, axis=1, keepdims=True) + bo_ref[...]

    row_spec = pl.BlockSpec((BLK, DF), lambda i: (i, 0))
    full = lambda a: pl.BlockSpec(a.shape, lambda i: (0,) * a.ndim)
    out = pl.pallas_call(
        body,
        grid=(grid,),
        in_specs=[row_spec, row_spec,
                  full(w1a), full(w1b), full(b1r), full(W2), full(b2r),
                  full(W3), full(b3r), full(womf_t), full(woh_t), full(bor)],
        out_specs=pl.BlockSpec((BLK, 1), lambda i: (i, 0)),
        out_shape=jax.ShapeDtypeStruct((B, 1), jnp.float32),
    )(u_rows, i_rows, w1a, w1b, b1r, W2, b2r, W3, b3r, womf_t, woh_t, bor)
    return out[:, 0]


def kernel(user_idx, item_idx, user_embedding_mf, item_embedding_mf,
           user_embedding_mlp, item_embedding_mlp, W1, b1, W2, b2, W3, b3,
           Wo, bo):
    D = user_embedding_mf.shape[1]
    tabs = (user_embedding_mf, item_embedding_mf,
            user_embedding_mlp, item_embedding_mlp)
    t_all = sum(jnp.pad(t, ((0, 0), (k * D, (3 - k) * D)))
                for k, t in enumerate(tabs))
    u_rows, i_rows = _sc_gather2(
        user_idx.astype(jnp.int32), item_idx.astype(jnp.int32), t_all)
    return _tc_mlp(u_rows, i_rows, W1, b1, W2, b2, W3, b3, Wo, bo)


# speedup vs baseline: 1.6775x; 1.3816x over previous
import functools

import jax
import jax.numpy as jnp
from jax import lax
from jax.experimental import pallas as pl
from jax.experimental.pallas import tpu as pltpu
from jax.experimental.pallas import tpu_sc as plsc


def _sc_gather2(uidx, iidx, t_all):
    B = uidx.shape[0]
    DF = t_all.shape[1]
    info = plsc.get_sparse_core_info()
    nc = info.num_cores
    nw = nc * info.num_subcores
    bpw = B // nw
    mesh = plsc.VectorSubcoreMesh(core_axis_name="c", subcore_axis_name="s")
    out_t = jax.ShapeDtypeStruct((B, DF), jnp.float32)

    @functools.partial(
        pl.kernel,
        mesh=mesh,
        out_type=[out_t, out_t],
        scratch_types=[
            pltpu.VMEM((bpw,), jnp.int32),
            pltpu.VMEM((bpw,), jnp.int32),
            pltpu.VMEM((bpw, DF), jnp.float32),
            pltpu.SemaphoreType.DMA,
        ],
    )
    def gather_kernel(uidx_hbm, iidx_hbm, tbl, o_u, o_i,
                      sidx_u, sidx_i, rows, gsem):
        wid = lax.axis_index("s") * nc + lax.axis_index("c")
        base = wid * bpw
        sl = pl.ds(base, bpw)
        pltpu.sync_copy(uidx_hbm.at[sl], sidx_u)
        pltpu.sync_copy(iidx_hbm.at[sl], sidx_i)

        def do_side(sidx, oref):
            def fire(j, _):
                v = sidx[pl.ds(j * 16, 16)]
                for l in range(16):
                    pltpu.async_copy(tbl.at[v[l]], rows.at[j * 16 + l], gsem)
                return _

            lax.fori_loop(0, bpw // 16, fire, None)

            def drain(j, _):
                pltpu.make_async_copy(tbl.at[0], rows.at[0], gsem).wait()
                return _

            lax.fori_loop(0, bpw, drain, None)
            pltpu.sync_copy(rows, oref.at[sl])

        do_side(sidx_u, o_u)
        do_side(sidx_i, o_i)

    return gather_kernel(uidx, iidx, t_all)


def _tc_mlp(u_rows, i_rows, W1, b1, W2, b2, W3, b3, Wo, bo):
    B, DF = u_rows.shape
    D = DF // 4
    BLK = 2048
    grid = B // BLK
    w1a, w1b = W1[:D], W1[D:]
    womf_t = Wo[:D].reshape(1, D)
    woh_t = Wo[D:].reshape(1, -1)
    b1r = b1.reshape(1, -1)
    b2r = b2.reshape(1, -1)
    b3r = b3.reshape(1, -1)
    bor = bo.reshape(1, 1)

    def body(u_ref, i_ref,
             w1a_ref, w1b_ref, b1_ref, w2_ref, b2_ref, w3_ref, b3_ref,
             womf_ref, woh_ref, bo_ref, out_ref):
        u = u_ref[...]
        i = i_ref[...]
        u_mf = u[:, 0:D]
        i_mf = i[:, D:2 * D]
        u_mlp = u[:, 2 * D:3 * D]
        i_mlp = i[:, 3 * D:4 * D]
        h = jnp.dot(u_mlp, w1a_ref[...], preferred_element_type=jnp.float32)
        h = h + jnp.dot(i_mlp, w1b_ref[...], preferred_element_type=jnp.float32)
        h = jnp.maximum(h + b1_ref[...], 0.0)
        h = jnp.dot(h, w2_ref[...], preferred_element_type=jnp.float32)
        h = jnp.maximum(h + b2_ref[...], 0.0)
        h = jnp.dot(h, w3_ref[...], preferred_element_type=jnp.float32)
        h = jnp.maximum(h + b3_ref[...], 0.0)
        mf = u_mf * i_mf
        acc = mf * womf_ref[...] + h * woh_ref[...]
        out_ref[...] = jnp.sum(acc, axis=1, keepdims=True) + bo_ref[...]

    row_spec = pl.BlockSpec((BLK, DF), lambda i: (i, 0))
    full = lambda a: pl.BlockSpec(a.shape, lambda i: (0,) * a.ndim)
    out = pl.pallas_call(
        body,
        grid=(grid,),
        in_specs=[row_spec, row_spec,
                  full(w1a), full(w1b), full(b1r), full(W2), full(b2r),
                  full(W3), full(b3r), full(womf_t), full(woh_t), full(bor)],
        out_specs=pl.BlockSpec((BLK, 1), lambda i: (i, 0)),
        out_shape=jax.ShapeDtypeStruct((B, 1), jnp.float32),
    )(u_rows, i_rows, w1a, w1b, b1r, W2, b2r, W3, b3r, womf_t, woh_t, bor)
    return out[:, 0]


def _tc_repack(umf_t, imf_t, umlp_t, imlp_t):
    D, V = umf_t.shape
    BLK = 2048
    grid = pl.cdiv(V, BLK)

    def body(a_ref, b_ref, c_ref, d_ref, out_ref):
        for k, r in enumerate((a_ref, b_ref, c_ref, d_ref)):
            out_ref[:, k * D:(k + 1) * D] = jnp.transpose(r[...])

    in_spec = pl.BlockSpec((D, BLK), lambda i: (0, i))
    return pl.pallas_call(
        body,
        grid=(grid,),
        in_specs=[in_spec, in_spec, in_spec, in_spec],
        out_specs=pl.BlockSpec((BLK, 4 * D), lambda i: (i, 0)),
        out_shape=jax.ShapeDtypeStruct((V, 4 * D), jnp.float32),
    )(umf_t, imf_t, umlp_t, imlp_t)


def kernel(user_idx, item_idx, user_embedding_mf, item_embedding_mf,
           user_embedding_mlp, item_embedding_mlp, W1, b1, W2, b2, W3, b3,
           Wo, bo):
    t_all = _tc_repack(user_embedding_mf.T, item_embedding_mf.T,
                       user_embedding_mlp.T, item_embedding_mlp.T)
    u_rows, i_rows = _sc_gather2(
        user_idx.astype(jnp.int32), item_idx.astype(jnp.int32), t_all)
    return _tc_mlp(u_rows, i_rows, W1, b1, W2, b2, W3, b3, Wo, bo)


# speedup vs baseline: 1.7238x; 1.0276x over previous
import functools

import jax
import jax.numpy as jnp
from jax import lax
from jax.experimental import pallas as pl
from jax.experimental.pallas import tpu as pltpu
from jax.experimental.pallas import tpu_sc as plsc


def _sc_gather2(uidx, iidx, t_all):
    B = uidx.shape[0]
    DF = t_all.shape[1]
    info = plsc.get_sparse_core_info()
    nc = info.num_cores
    nw = nc * info.num_subcores
    bpw = B // nw
    mesh = plsc.VectorSubcoreMesh(core_axis_name="c", subcore_axis_name="s")
    out_t = jax.ShapeDtypeStruct((B, DF), jnp.float32)

    @functools.partial(
        pl.kernel,
        mesh=mesh,
        out_type=[out_t, out_t],
        scratch_types=[
            pltpu.VMEM((bpw,), jnp.int32),
            pltpu.VMEM((bpw,), jnp.int32),
            pltpu.VMEM((bpw, DF), jnp.float32),
            pltpu.SemaphoreType.DMA,
        ],
    )
    def gather_kernel(uidx_hbm, iidx_hbm, tbl, o_u, o_i,
                      sidx_u, sidx_i, rows, gsem):
        wid = lax.axis_index("s") * nc + lax.axis_index("c")
        base = wid * bpw
        sl = pl.ds(base, bpw)
        pltpu.sync_copy(uidx_hbm.at[sl], sidx_u)
        pltpu.sync_copy(iidx_hbm.at[sl], sidx_i)

        def do_side(sidx, oref):
            def fire(j, _):
                v = sidx[pl.ds(j * 16, 16)]
                for l in range(16):
                    pltpu.async_copy(tbl.at[v[l]], rows.at[j * 16 + l], gsem)
                return _

            lax.fori_loop(0, bpw // 16, fire, None)

            def drain(j, _):
                pltpu.make_async_copy(tbl.at[0], rows.at[0], gsem).wait()
                return _

            lax.fori_loop(0, bpw, drain, None)
            pltpu.sync_copy(rows, oref.at[sl])

        do_side(sidx_u, o_u)
        do_side(sidx_i, o_i)

    return gather_kernel(uidx, iidx, t_all)


def _tc_mlp(u_rows, i_rows, W1, b1, W2, b2, W3, b3, Wo, bo):
    B, DF = u_rows.shape
    D = DF // 4
    BLK = 2048
    grid = B // BLK
    w1a, w1b = W1[:D], W1[D:]
    womf_t = Wo[:D].reshape(1, D)
    woh_t = Wo[D:].reshape(1, -1)
    b1r = b1.reshape(1, -1)
    b2r = b2.reshape(1, -1)
    b3r = b3.reshape(1, -1)
    bor = bo.reshape(1, 1)

    def body(u_ref, i_ref,
             w1a_ref, w1b_ref, b1_ref, w2_ref, b2_ref, w3_ref, b3_ref,
             womf_ref, woh_ref, bo_ref, out_ref):
        u = u_ref[...]
        i = i_ref[...]
        u_mf = u[:, 0:D]
        i_mf = i[:, D:2 * D]
        u_mlp = u[:, 2 * D:3 * D]
        i_mlp = i[:, 3 * D:4 * D]
        h = jnp.dot(u_mlp, w1a_ref[...], preferred_element_type=jnp.float32)
        h = h + jnp.dot(i_mlp, w1b_ref[...], preferred_element_type=jnp.float32)
        h = jnp.maximum(h + b1_ref[...], 0.0)
        h = jnp.dot(h, w2_ref[...], preferred_element_type=jnp.float32)
        h = jnp.maximum(h + b2_ref[...], 0.0)
        h = jnp.dot(h, w3_ref[...], preferred_element_type=jnp.float32)
        h = jnp.maximum(h + b3_ref[...], 0.0)
        mf = u_mf * i_mf
        acc = mf * womf_ref[...] + h * woh_ref[...]
        out_ref[...] = jnp.sum(acc, axis=1, keepdims=True) + bo_ref[...]

    row_spec = pl.BlockSpec((BLK, DF), lambda i: (i, 0))
    full = lambda a: pl.BlockSpec(a.shape, lambda i: (0,) * a.ndim)
    out = pl.pallas_call(
        body,
        grid=(grid,),
        in_specs=[row_spec, row_spec,
                  full(w1a), full(w1b), full(b1r), full(W2), full(b2r),
                  full(W3), full(b3r), full(womf_t), full(woh_t), full(bor)],
        out_specs=pl.BlockSpec((BLK, 1), lambda i: (i, 0)),
        out_shape=jax.ShapeDtypeStruct((B, 1), jnp.float32),
    )(u_rows, i_rows, w1a, w1b, b1r, W2, b2r, W3, b3r, womf_t, woh_t, bor)
    return out[:, 0]


def _tc_repack(umf_t, imf_t, umlp_t, imlp_t):
    D, V = umf_t.shape
    BLK = 4096
    grid = pl.cdiv(V, BLK)

    def body(a_ref, b_ref, c_ref, d_ref, out_ref):
        out_ref[...] = jnp.concatenate(
            [jnp.transpose(r[...]) for r in (a_ref, b_ref, c_ref, d_ref)],
            axis=1)

    in_spec = pl.BlockSpec((D, BLK), lambda i: (0, i))
    return pl.pallas_call(
        body,
        grid=(grid,),
        in_specs=[in_spec, in_spec, in_spec, in_spec],
        out_specs=pl.BlockSpec((BLK, 4 * D), lambda i: (i, 0)),
        out_shape=jax.ShapeDtypeStruct((V, 4 * D), jnp.float32),
    )(umf_t, imf_t, umlp_t, imlp_t)


def kernel(user_idx, item_idx, user_embedding_mf, item_embedding_mf,
           user_embedding_mlp, item_embedding_mlp, W1, b1, W2, b2, W3, b3,
           Wo, bo):
    t_all = _tc_repack(user_embedding_mf.T, item_embedding_mf.T,
                       user_embedding_mlp.T, item_embedding_mlp.T)
    u_rows, i_rows = _sc_gather2(
        user_idx.astype(jnp.int32), item_idx.astype(jnp.int32), t_all)
    return _tc_mlp(u_rows, i_rows, W1, b1, W2, b2, W3, b3, Wo, bo)
